# baseline (speedup 1.0000x reference)
import functools

import jax
import jax.numpy as jnp
from jax.experimental import pallas as pl
from jax.experimental.pallas import tpu as pltpu

N = 4096
D = 256
BR = 512


def _mm_kernel(a_ref, x_ref, o_ref):
    o_ref[...] = jnp.dot(
        a_ref[...].astype(jnp.bfloat16),
        x_ref[...],
        preferred_element_type=jnp.float32,
    )


def _mm_add_kernel(a_ref, x_ref, r_ref, o_ref):
    o_ref[...] = (
        jnp.dot(
            a_ref[...].astype(jnp.bfloat16),
            x_ref[...],
            preferred_element_type=jnp.float32,
        )
        + r_ref[...]
    )


def _mm_mean_kernel(a_ref, x_ref, x0_ref, x1_ref, x2_ref, o_ref):
    dot = jnp.dot(
        a_ref[...].astype(jnp.bfloat16),
        x_ref[...],
        preferred_element_type=jnp.float32,
    )
    o_ref[...] = 0.25 * (x0_ref[...] + x1_ref[...] + dot) + 0.5 * x2_ref[...]


_row_spec = pl.BlockSpec((BR, N), lambda i: (i, 0))
_full_spec = pl.BlockSpec((N, D), lambda i: (0, 0))
_out_spec = pl.BlockSpec((BR, D), lambda i: (i, 0))
_params = pltpu.CompilerParams(dimension_semantics=("arbitrary",))


def _mm(a, x):
    return pl.pallas_call(
        _mm_kernel,
        grid=(N // BR,),
        in_specs=[_row_spec, _full_spec],
        out_specs=_out_spec,
        out_shape=jax.ShapeDtypeStruct((N, D), jnp.float32),
        compiler_params=_params,
    )(a, x)


def _mm_add(a, x, r):
    return pl.pallas_call(
        _mm_add_kernel,
        grid=(N // BR,),
        in_specs=[_row_spec, _full_spec, _out_spec],
        out_specs=_out_spec,
        out_shape=jax.ShapeDtypeStruct((N, D), jnp.float32),
        compiler_params=_params,
    )(a, x, r)


def _mm_mean(a, x, x0, x1, x2):
    return pl.pallas_call(
        _mm_mean_kernel,
        grid=(N // BR,),
        in_specs=[_row_spec, _full_spec, _out_spec, _out_spec, _out_spec],
        out_specs=_out_spec,
        out_shape=jax.ShapeDtypeStruct((N, D), jnp.float32),
        compiler_params=_params,
    )(a, x, x0, x1, x2)


def kernel(pois_embs, HG_poi_src, HG_poi_tar):
    x0 = pois_embs
    x0b = x0.astype(jnp.bfloat16)

    y1 = _mm(HG_poi_tar, x0b)
    x1 = _mm_add(HG_poi_src, y1.astype(jnp.bfloat16), x0)

    y2 = _mm(HG_poi_tar, x1.astype(jnp.bfloat16))
    x2 = _mm_add(HG_poi_src, y2.astype(jnp.bfloat16), x1)

    y3 = _mm(HG_poi_tar, x2.astype(jnp.bfloat16))
    return _mm_mean(HG_poi_src, y3.astype(jnp.bfloat16), x0, x1, x2)
